# baseline (speedup 1.0000x reference)
import functools

import jax
import jax.numpy as jnp
from jax import lax
from jax.experimental import pallas as pl
from jax.experimental.pallas import tpu as pltpu
from jax.experimental.pallas import tpu_sc as plsc

D_MODEL = 1024
HIDDEN = 4096
NUM_EXPERTS = 16
EPS = 1e-6

N_TOK = 8192
N_PAIR = 2 * N_TOK
BT = 1024
BTA = 256
TM = 256
P_SLOTS = N_PAIR + NUM_EXPERTS * TM
NT = P_SLOTS // TM

NW = 32
G_CHUNK = 40
C_CHUNK = 32


def _route_kernel(x_ref, rmsw_ref, gate_ref, tok_ref, eidx_ref, w_ref,
                  rank_ref, counts_ref, base_ref, te_ref, run_cnt):
    t = pl.program_id(0)

    @pl.when(t == 0)
    def _init():
        run_cnt[...] = jnp.zeros_like(run_cnt)

    x = x_ref[...]
    nrm = x * lax.rsqrt(jnp.mean(jnp.square(x), axis=-1, keepdims=True) + EPS)
    nrm = nrm * rmsw_ref[...][None, :]
    tok_ref[...] = nrm

    logits = jnp.dot(nrm, gate_ref[...].T, preferred_element_type=jnp.float32)
    m0 = jnp.max(logits, axis=-1, keepdims=True)
    i0 = jnp.argmax(logits, axis=-1).astype(jnp.int32)
    col = lax.broadcasted_iota(jnp.int32, logits.shape, 1)
    masked = jnp.where(col == i0[:, None], -jnp.inf, logits)
    m1 = jnp.max(masked, axis=-1, keepdims=True)
    i1 = jnp.argmax(masked, axis=-1).astype(jnp.int32)
    w0 = 1.0 / (1.0 + jnp.exp(m1 - m0))
    w1 = 1.0 - w0
    eidx_ref[...] = jnp.concatenate([i0[:, None], i1[:, None]], axis=1)
    w_ref[...] = jnp.concatenate([w0, w1], axis=1)

    oh0 = (col == i0[:, None]).astype(jnp.float32)
    oh1 = (col == i1[:, None]).astype(jnp.float32)
    c = oh0 + oh1
    row = lax.broadcasted_iota(jnp.int32, (BTA, BTA), 0)
    cidx = lax.broadcasted_iota(jnp.int32, (BTA, BTA), 1)
    lt = (cidx < row).astype(jnp.float32)
    excl = jnp.dot(lt, c, preferred_element_type=jnp.float32)
    run = run_cnt[...].astype(jnp.float32)
    tot = excl + run
    rank0 = jnp.sum(tot * oh0, axis=1, keepdims=True)
    rank1 = jnp.sum(tot * oh1, axis=1, keepdims=True)
    rank_ref[...] = jnp.concatenate([rank0, rank1], axis=1).astype(jnp.int32)
    run_cnt[...] += jnp.sum(c, axis=0, keepdims=True).astype(jnp.int32)

    cnt = run_cnt[...]
    counts_ref[...] = cnt
    padded = ((cnt + (TM - 1)) // TM) * TM
    pr = lax.broadcasted_iota(jnp.int32, (NUM_EXPERTS, NUM_EXPERTS), 0)
    pc = lax.broadcasted_iota(jnp.int32, (NUM_EXPERTS, NUM_EXPERTS), 1)
    contrib = jnp.where(pr < pc, jnp.broadcast_to(
        padded.reshape(NUM_EXPERTS, 1), (NUM_EXPERTS, NUM_EXPERTS)), 0)
    base = jnp.sum(contrib, axis=0).reshape(1, NUM_EXPERTS)
    base_ref[...] = base
    ends = (base + padded).reshape(NUM_EXPERTS, 1)
    jt = lax.broadcasted_iota(jnp.int32, (NUM_EXPERTS, NT), 1) * TM
    te = jnp.sum((ends <= jt).astype(jnp.int32), axis=0).reshape(1, NT)
    te_ref[...] = jnp.minimum(te, NUM_EXPERTS - 1)


def _route(xf, rms_weight, gate_w):
    return pl.pallas_call(
        _route_kernel,
        grid=(N_TOK // BTA,),
        in_specs=[
            pl.BlockSpec((BTA, D_MODEL), lambda t: (t, 0)),
            pl.BlockSpec((D_MODEL,), lambda t: (0,)),
            pl.BlockSpec((NUM_EXPERTS, D_MODEL), lambda t: (0, 0)),
        ],
        out_specs=[
            pl.BlockSpec((BTA, D_MODEL), lambda t: (t, 0)),
            pl.BlockSpec((BTA, 2), lambda t: (t, 0)),
            pl.BlockSpec((BTA, 2), lambda t: (t, 0)),
            pl.BlockSpec((BTA, 2), lambda t: (t, 0)),
            pl.BlockSpec((1, NUM_EXPERTS), lambda t: (0, 0)),
            pl.BlockSpec((1, NUM_EXPERTS), lambda t: (0, 0)),
            pl.BlockSpec((1, NT), lambda t: (0, 0)),
        ],
        out_shape=[
            jax.ShapeDtypeStruct((N_TOK, D_MODEL), jnp.float32),
            jax.ShapeDtypeStruct((N_TOK, 2), jnp.int32),
            jax.ShapeDtypeStruct((N_TOK, 2), jnp.float32),
            jax.ShapeDtypeStruct((N_TOK, 2), jnp.int32),
            jax.ShapeDtypeStruct((1, NUM_EXPERTS), jnp.int32),
            jax.ShapeDtypeStruct((1, NUM_EXPERTS), jnp.int32),
            jax.ShapeDtypeStruct((1, NT), jnp.int32),
        ],
        scratch_shapes=[pltpu.VMEM((1, NUM_EXPERTS), jnp.int32)],
    )(xf, rms_weight, gate_w)


N_PW = N_PAIR // NW


def _dispatch_body(eidx_hbm, rank_hbm, w_hbm, base_hbm,
                   pos_hbm, gtok_hbm, gw_hbm,
                   ve, vr, vw, vbase, vpos, vtok, sem):
    cid = lax.axis_index("c")
    sid = lax.axis_index("s")
    wid = sid * 2 + cid
    off = wid * N_PW

    pltpu.sync_copy(eidx_hbm.at[pl.ds(off, N_PW)], ve)
    pltpu.sync_copy(rank_hbm.at[pl.ds(off, N_PW)], vr)
    pltpu.sync_copy(w_hbm.at[pl.ds(off, N_PW)], vw)
    pltpu.sync_copy(base_hbm, vbase)

    lane = lax.iota(jnp.int32, 16)

    def body(g, _):
        ev = ve[pl.ds(g * 16, 16)]
        rr = vr[pl.ds(g * 16, 16)]
        bb = plsc.load_gather(vbase, [ev])
        vpos[pl.ds(g * 16, 16)] = rr + bb
        vtok[pl.ds(g * 16, 16)] = jnp.right_shift(off + g * 16 + lane, 1)
        return 0

    lax.fori_loop(0, N_PW // 16, body, 0)

    pltpu.sync_copy(vpos, pos_hbm.at[pl.ds(off, N_PW)])
    pltpu.async_copy(vtok, gtok_hbm.at[vpos], sem).wait()
    pltpu.async_copy(vw, gw_hbm.at[vpos], sem).wait()


def _dispatch(eidx_f, rank_f, w_f, base_f):
    mesh = plsc.VectorSubcoreMesh(core_axis_name="c", subcore_axis_name="s")
    fn = pl.kernel(
        _dispatch_body,
        out_type=[
            jax.ShapeDtypeStruct((N_PAIR,), jnp.int32),
            jax.ShapeDtypeStruct((P_SLOTS,), jnp.int32),
            jax.ShapeDtypeStruct((P_SLOTS,), jnp.float32),
        ],
        mesh=mesh,
        scratch_types=[
            pltpu.VMEM((N_PW,), jnp.int32),
            pltpu.VMEM((N_PW,), jnp.int32),
            pltpu.VMEM((N_PW,), jnp.float32),
            pltpu.VMEM((NUM_EXPERTS,), jnp.int32),
            pltpu.VMEM((N_PW,), jnp.int32),
            pltpu.VMEM((N_PW,), jnp.int32),
            pltpu.SemaphoreType.DMA,
        ],
        compiler_params=pltpu.CompilerParams(needs_layout_passes=False),
    )
    return fn(eidx_f, rank_f, w_f, base_f)


def _make_gather(n_rows, n_chunk, dtype, sl=8, clamp_max=None):
    rows_per_w = n_rows // NW
    n_loops = rows_per_w // n_chunk
    assert rows_per_w % n_chunk == 0 and n_loops >= 2

    def body(idx_hbm, table_hbm, out_hbm, idx_v, rows0, rows1,
             gs0, gs1, os0, os1):
        cid = lax.axis_index("c")
        sid = lax.axis_index("s")
        wid = sid * 2 + cid
        base = wid * rows_per_w
        pltpu.sync_copy(idx_hbm.at[pl.ds(base, rows_per_w)], idx_v)
        if clamp_max is not None:
            hi = jnp.full((16,), clamp_max, jnp.int32)
            lo = jnp.zeros((16,), jnp.int32)

            def cbody(j, _):
                v = idx_v[pl.ds(j * 16, 16)]
                idx_v[pl.ds(j * 16, 16)] = jnp.minimum(jnp.maximum(v, lo), hi)
                return 0

            lax.fori_loop(0, rows_per_w // 16, cbody, 0)

        bufs = (rows0, rows1)
        gsems = (gs0, gs1)
        osems = (os0, os1)
        gd = [None] * n_loops
        od = [None] * n_loops

        def issue_gather(ci):
            return pltpu.async_copy(
                table_hbm.at[idx_v.at[pl.ds(ci * n_chunk, n_chunk)]],
                bufs[ci % 2], gsems[ci % 2])

        gd[0] = issue_gather(0)
        gd[1] = issue_gather(1)
        for ci in range(n_loops):
            gd[ci].wait()
            od[ci] = pltpu.async_copy(
                bufs[ci % 2],
                out_hbm.at[pl.ds(base + ci * n_chunk, n_chunk)],
                osems[ci % 2])
            nxt = ci + 2
            if nxt < n_loops:
                od[ci].wait()
                gd[nxt] = issue_gather(nxt)
        od[n_loops - 2].wait()
        od[n_loops - 1].wait()

    mesh = plsc.VectorSubcoreMesh(core_axis_name="c", subcore_axis_name="s")
    return pl.kernel(
        body,
        out_type=jax.ShapeDtypeStruct((n_rows, sl, 128), dtype),
        mesh=mesh,
        scratch_types=[
            pltpu.VMEM((rows_per_w,), jnp.int32),
            pltpu.VMEM((n_chunk, sl, 128), dtype),
            pltpu.VMEM((n_chunk, sl, 128), dtype),
            pltpu.SemaphoreType.DMA,
            pltpu.SemaphoreType.DMA,
            pltpu.SemaphoreType.DMA,
            pltpu.SemaphoreType.DMA,
        ],
        compiler_params=pltpu.CompilerParams(needs_layout_passes=False),
    )


def _cast_kernel(fc1_ref, fc2_ref, o1_ref, o2_ref):
    o1_ref[...] = fc1_ref[...].astype(jnp.bfloat16)
    o2_ref[...] = fc2_ref[...].astype(jnp.bfloat16)


def _cast_weights(fc1_w, fc2_w):
    nh = HIDDEN // D_MODEL
    return pl.pallas_call(
        _cast_kernel,
        grid=(NUM_EXPERTS, nh),
        in_specs=[
            pl.BlockSpec((1, D_MODEL, D_MODEL), lambda e, h: (e, h, 0)),
            pl.BlockSpec((1, D_MODEL, D_MODEL), lambda e, h: (e, 0, h)),
        ],
        out_specs=[
            pl.BlockSpec((1, D_MODEL, D_MODEL), lambda e, h: (e, h, 0)),
            pl.BlockSpec((1, D_MODEL, D_MODEL), lambda e, h: (e, 0, h)),
        ],
        out_shape=[
            jax.ShapeDtypeStruct((NUM_EXPERTS, HIDDEN, D_MODEL), jnp.bfloat16),
            jax.ShapeDtypeStruct((NUM_EXPERTS, D_MODEL, HIDDEN), jnp.bfloat16),
        ],
    )(fc1_w, fc2_w)


def _ffn_kernel(te_ref, gtok_ref, gw_ref, fc1_ref, fc1b_ref, fc2_ref,
                fc2b_ref, out_ref):
    toks = gtok_ref[...].astype(jnp.bfloat16)
    hid = jnp.dot(toks, fc1_ref[0].T, preferred_element_type=jnp.float32)
    hid = hid + fc1b_ref[0]
    hid = 0.5 * hid * (1.0 + lax.erf(hid * 0.7071067811865476))
    o = jnp.dot(hid.astype(jnp.bfloat16), fc2_ref[0].T,
                preferred_element_type=jnp.float32)
    o = o + fc2b_ref[0]
    out_ref[...] = o * gw_ref[...]


def _ffn(te, gtoks, gw, fc1_bf, fc1_b3, fc2_bf, fc2_b3):
    grid_spec = pltpu.PrefetchScalarGridSpec(
        num_scalar_prefetch=1,
        grid=(NT,),
        in_specs=[
            pl.BlockSpec((TM, D_MODEL), lambda t, te: (t, 0)),
            pl.BlockSpec((TM, 1), lambda t, te: (t, 0)),
            pl.BlockSpec((1, HIDDEN, D_MODEL), lambda t, te: (te[t], 0, 0)),
            pl.BlockSpec((1, 1, HIDDEN), lambda t, te: (te[t], 0, 0)),
            pl.BlockSpec((1, D_MODEL, HIDDEN), lambda t, te: (te[t], 0, 0)),
            pl.BlockSpec((1, 1, D_MODEL), lambda t, te: (te[t], 0, 0)),
        ],
        out_specs=pl.BlockSpec((TM, D_MODEL), lambda t, te: (t, 0)),
    )
    return pl.pallas_call(
        _ffn_kernel,
        grid_spec=grid_spec,
        out_shape=jax.ShapeDtypeStruct((P_SLOTS, D_MODEL), jnp.float32),
    )(te, gtoks, gw, fc1_bf, fc1_b3, fc2_bf, fc2_b3)


def _combine_kernel(x_ref, op_ref, out_ref):
    out_ref[...] = x_ref[...] + op_ref[:, 0, :] + op_ref[:, 1, :]


def _combine(xf, op_tok3):
    return pl.pallas_call(
        _combine_kernel,
        grid=(N_TOK // BT,),
        in_specs=[
            pl.BlockSpec((BT, D_MODEL), lambda t: (t, 0)),
            pl.BlockSpec((BT, 2, D_MODEL), lambda t: (t, 0, 0)),
        ],
        out_specs=pl.BlockSpec((BT, D_MODEL), lambda t: (t, 0)),
        out_shape=jax.ShapeDtypeStruct((N_TOK, D_MODEL), jnp.float32),
    )(xf, op_tok3)


def kernel(x, rms_weight, gate_w, fc1_w, fc1_b, fc2_w, fc2_b):
    b, s, d = x.shape
    xf = x.reshape(N_TOK, d)
    fc1_bf, fc2_bf = _cast_weights(fc1_w, fc2_w)
    fc1_b3 = fc1_b.reshape(NUM_EXPERTS, 1, HIDDEN)
    fc2_b3 = fc2_b.reshape(NUM_EXPERTS, 1, D_MODEL)

    tok_f, eidx, wpair, rank, _counts, base, te = _route(
        xf, rms_weight, gate_w)

    pos, gtok, gw = _dispatch(eidx.reshape(N_PAIR), rank.reshape(N_PAIR),
                              wpair.reshape(N_PAIR), base.reshape(NUM_EXPERTS))

    gtoks3 = _make_gather(P_SLOTS, G_CHUNK, jnp.float32, sl=8,
                          clamp_max=N_TOK - 1)(
        gtok, tok_f.reshape(N_TOK, 8, 128))

    out_pairs = _ffn(te.reshape(NT), gtoks3.reshape(P_SLOTS, D_MODEL),
                     gw.reshape(P_SLOTS, 1), fc1_bf, fc1_b3, fc2_bf, fc2_b3)

    op_tok = _make_gather(N_PAIR, C_CHUNK, jnp.float32)(
        pos, out_pairs.reshape(P_SLOTS, 8, 128))

    out = _combine(xf, op_tok.reshape(N_TOK, 2, D_MODEL))
    return out.reshape(b, s, d)


# speedup vs baseline: 1.3039x; 1.3039x over previous
import functools

import jax
import jax.numpy as jnp
from jax import lax
from jax.experimental import pallas as pl
from jax.experimental.pallas import tpu as pltpu
from jax.experimental.pallas import tpu_sc as plsc

D_MODEL = 1024
HIDDEN = 4096
NUM_EXPERTS = 16
EPS = 1e-6

N_TOK = 8192
N_PAIR = 2 * N_TOK
BT = 1024
BTA = 256
TM = 256
P_SLOTS = N_PAIR + NUM_EXPERTS * TM
NT = P_SLOTS // TM

NW = 32
NTW = N_TOK // NW
D_CHUNK = 32
C_CHUNK = 32


def _route_kernel(x_ref, rmsw_ref, gate_ref, tok_ref, eidx_ref, rank_ref,
                  w_ref, base_ref, te_ref, run_cnt):
    t = pl.program_id(0)

    @pl.when(t == 0)
    def _init():
        run_cnt[...] = jnp.zeros_like(run_cnt)

    x = x_ref[...]
    nrm = x * lax.rsqrt(jnp.mean(jnp.square(x), axis=-1, keepdims=True) + EPS)
    nrm = nrm * rmsw_ref[...][None, :]
    tok_ref[...] = nrm

    logits = jnp.dot(nrm, gate_ref[...].T, preferred_element_type=jnp.float32)
    m0 = jnp.max(logits, axis=-1, keepdims=True)
    i0 = jnp.argmax(logits, axis=-1).astype(jnp.int32)
    col = lax.broadcasted_iota(jnp.int32, logits.shape, 1)
    masked = jnp.where(col == i0[:, None], -jnp.inf, logits)
    m1 = jnp.max(masked, axis=-1, keepdims=True)
    i1 = jnp.argmax(masked, axis=-1).astype(jnp.int32)
    w0 = 1.0 / (1.0 + jnp.exp(m1 - m0))
    w1 = 1.0 - w0
    eidx_ref[...] = jnp.concatenate(
        [i0.reshape(1, BTA), i1.reshape(1, BTA)], axis=0)
    w_ref[...] = jnp.concatenate([w0, w1], axis=1)

    oh0 = (col == i0[:, None]).astype(jnp.float32)
    oh1 = (col == i1[:, None]).astype(jnp.float32)
    c = oh0 + oh1
    row = lax.broadcasted_iota(jnp.int32, (BTA, BTA), 0)
    cidx = lax.broadcasted_iota(jnp.int32, (BTA, BTA), 1)
    lt = (cidx < row).astype(jnp.float32)
    excl = jnp.dot(lt, c, preferred_element_type=jnp.float32)
    run = run_cnt[...].astype(jnp.float32)
    tot = excl + run
    rank0 = jnp.sum(tot * oh0, axis=1, keepdims=True).astype(jnp.int32)
    rank1 = jnp.sum(tot * oh1, axis=1, keepdims=True).astype(jnp.int32)
    rank_ref[...] = jnp.concatenate(
        [rank0.reshape(1, BTA), rank1.reshape(1, BTA)], axis=0)
    run_cnt[...] += jnp.sum(c, axis=0, keepdims=True).astype(jnp.int32)

    cnt = run_cnt[...]
    padded = ((cnt + (TM - 1)) // TM) * TM
    pr = lax.broadcasted_iota(jnp.int32, (NUM_EXPERTS, NUM_EXPERTS), 0)
    pc = lax.broadcasted_iota(jnp.int32, (NUM_EXPERTS, NUM_EXPERTS), 1)
    contrib = jnp.where(pr < pc, jnp.broadcast_to(
        padded.reshape(NUM_EXPERTS, 1), (NUM_EXPERTS, NUM_EXPERTS)), 0)
    base = jnp.sum(contrib, axis=0).reshape(1, NUM_EXPERTS)
    base_ref[...] = base
    ends = (base + padded).reshape(NUM_EXPERTS, 1)
    jt = lax.broadcasted_iota(jnp.int32, (NUM_EXPERTS, NT), 1) * TM
    te = jnp.sum((ends <= jt).astype(jnp.int32), axis=0).reshape(1, NT)
    te_ref[...] = jnp.minimum(te, NUM_EXPERTS - 1)


def _route(xf, rms_weight, gate_w):
    return pl.pallas_call(
        _route_kernel,
        grid=(N_TOK // BTA,),
        in_specs=[
            pl.BlockSpec((BTA, D_MODEL), lambda t: (t, 0)),
            pl.BlockSpec((D_MODEL,), lambda t: (0,)),
            pl.BlockSpec((NUM_EXPERTS, D_MODEL), lambda t: (0, 0)),
        ],
        out_specs=[
            pl.BlockSpec((BTA, D_MODEL), lambda t: (t, 0)),
            pl.BlockSpec((2, BTA), lambda t: (0, t)),
            pl.BlockSpec((2, BTA), lambda t: (0, t)),
            pl.BlockSpec((BTA, 2), lambda t: (t, 0)),
            pl.BlockSpec((1, NUM_EXPERTS), lambda t: (0, 0)),
            pl.BlockSpec((1, NT), lambda t: (0, 0)),
        ],
        out_shape=[
            jax.ShapeDtypeStruct((N_TOK, D_MODEL), jnp.float32),
            jax.ShapeDtypeStruct((2, N_TOK), jnp.int32),
            jax.ShapeDtypeStruct((2, N_TOK), jnp.int32),
            jax.ShapeDtypeStruct((N_TOK, 2), jnp.float32),
            jax.ShapeDtypeStruct((1, NUM_EXPERTS), jnp.int32),
            jax.ShapeDtypeStruct((1, NT), jnp.int32),
        ],
        scratch_shapes=[pltpu.VMEM((1, NUM_EXPERTS), jnp.int32)],
    )(xf, rms_weight, gate_w)


_N_DCHUNKS = NTW // D_CHUNK


def _dispatch_body(eidx_hbm, rank_hbm, base_hbm, tok_hbm,
                   pos_hbm, grows_hbm,
                   ve0, ve1, vr0, vr1, vbase, vp0, vp1, vp0c, vp1c,
                   row0, row1, ls0, ls1, ss0, ss1):
    cid = lax.axis_index("c")
    sid = lax.axis_index("s")
    wid = sid * 2 + cid
    toff = wid * NTW

    pltpu.sync_copy(eidx_hbm.at[0, pl.ds(toff, NTW)], ve0)
    pltpu.sync_copy(eidx_hbm.at[1, pl.ds(toff, NTW)], ve1)
    pltpu.sync_copy(rank_hbm.at[0, pl.ds(toff, NTW)], vr0)
    pltpu.sync_copy(rank_hbm.at[1, pl.ds(toff, NTW)], vr1)
    pltpu.sync_copy(base_hbm, vbase)

    for g in range(NTW // 16):
        c, o = g // (D_CHUNK // 16), (g % (D_CHUNK // 16)) * 16
        p0 = vr0[pl.ds(g * 16, 16)] + plsc.load_gather(
            vbase, [ve0[pl.ds(g * 16, 16)]])
        vp0[pl.ds(g * 16, 16)] = p0
        vp0c[c, pl.ds(o, 16)] = p0
        p1 = vr1[pl.ds(g * 16, 16)] + plsc.load_gather(
            vbase, [ve1[pl.ds(g * 16, 16)]])
        vp1[pl.ds(g * 16, 16)] = p1
        vp1c[c, pl.ds(o, 16)] = p1

    pltpu.sync_copy(vp0, pos_hbm.at[0, pl.ds(toff, NTW)])
    pltpu.sync_copy(vp1, pos_hbm.at[1, pl.ds(toff, NTW)])

    bufs = (row0, row1)
    lsems = (ls0, ls1)
    ssems = (ss0, ss1)

    def load(c):
        return pltpu.async_copy(
            tok_hbm.at[pl.ds(toff + c * D_CHUNK, D_CHUNK)],
            bufs[c % 2], lsems[c % 2])

    ld = [None] * _N_DCHUNKS
    sc = [None] * _N_DCHUNKS
    ld[0] = load(0)
    ld[1] = load(1)
    for c in range(_N_DCHUNKS):
        ld[c].wait()
        s0 = pltpu.async_copy(bufs[c % 2], grows_hbm.at[vp0c.at[c]],
                              ssems[c % 2])
        s1 = pltpu.async_copy(bufs[c % 2], grows_hbm.at[vp1c.at[c]],
                              ssems[c % 2])
        sc[c] = (s0, s1)
        nxt = c + 2
        if nxt < _N_DCHUNKS:
            sc[c][0].wait()
            sc[c][1].wait()
            ld[nxt] = load(nxt)
    for c in (_N_DCHUNKS - 2, _N_DCHUNKS - 1):
        sc[c][0].wait()
        sc[c][1].wait()


def _dispatch(eidx_k, rank_k, base_f, tok3):
    mesh = plsc.VectorSubcoreMesh(core_axis_name="c", subcore_axis_name="s")
    fn = pl.kernel(
        _dispatch_body,
        out_type=[
            jax.ShapeDtypeStruct((2, N_TOK), jnp.int32),
            jax.ShapeDtypeStruct((P_SLOTS, 8, 128), jnp.float32),
        ],
        mesh=mesh,
        scratch_types=[
            pltpu.VMEM((NTW,), jnp.int32),
            pltpu.VMEM((NTW,), jnp.int32),
            pltpu.VMEM((NTW,), jnp.int32),
            pltpu.VMEM((NTW,), jnp.int32),
            pltpu.VMEM((NUM_EXPERTS,), jnp.int32),
            pltpu.VMEM((NTW,), jnp.int32),
            pltpu.VMEM((NTW,), jnp.int32),
            pltpu.VMEM((_N_DCHUNKS, D_CHUNK), jnp.int32),
            pltpu.VMEM((_N_DCHUNKS, D_CHUNK), jnp.int32),
            pltpu.VMEM((D_CHUNK, 8, 128), jnp.float32),
            pltpu.VMEM((D_CHUNK, 8, 128), jnp.float32),
            pltpu.SemaphoreType.DMA,
            pltpu.SemaphoreType.DMA,
            pltpu.SemaphoreType.DMA,
            pltpu.SemaphoreType.DMA,
        ],
        compiler_params=pltpu.CompilerParams(needs_layout_passes=False),
    )
    return fn(eidx_k, rank_k, base_f, tok3)


def _make_gather(n_rows, n_chunk, dtype, sl=8):
    rows_per_w = n_rows // NW
    n_loops = rows_per_w // n_chunk
    assert rows_per_w % n_chunk == 0 and n_loops >= 2

    def body(idx_hbm, table_hbm, out_hbm, idx_v, rows0, rows1,
             gs0, gs1, os0, os1):
        cid = lax.axis_index("c")
        sid = lax.axis_index("s")
        wid = sid * 2 + cid
        base = wid * rows_per_w
        pltpu.sync_copy(idx_hbm.at[pl.ds(base, rows_per_w)], idx_v)

        bufs = (rows0, rows1)
        gsems = (gs0, gs1)
        osems = (os0, os1)
        gd = [None] * n_loops
        od = [None] * n_loops

        def issue_gather(ci):
            return pltpu.async_copy(
                table_hbm.at[idx_v.at[pl.ds(ci * n_chunk, n_chunk)]],
                bufs[ci % 2], gsems[ci % 2])

        gd[0] = issue_gather(0)
        gd[1] = issue_gather(1)
        for ci in range(n_loops):
            gd[ci].wait()
            od[ci] = pltpu.async_copy(
                bufs[ci % 2],
                out_hbm.at[pl.ds(base + ci * n_chunk, n_chunk)],
                osems[ci % 2])
            nxt = ci + 2
            if nxt < n_loops:
                od[ci].wait()
                gd[nxt] = issue_gather(nxt)
        od[n_loops - 2].wait()
        od[n_loops - 1].wait()

    mesh = plsc.VectorSubcoreMesh(core_axis_name="c", subcore_axis_name="s")
    return pl.kernel(
        body,
        out_type=jax.ShapeDtypeStruct((n_rows, sl, 128), dtype),
        mesh=mesh,
        scratch_types=[
            pltpu.VMEM((rows_per_w,), jnp.int32),
            pltpu.VMEM((n_chunk, sl, 128), dtype),
            pltpu.VMEM((n_chunk, sl, 128), dtype),
            pltpu.SemaphoreType.DMA,
            pltpu.SemaphoreType.DMA,
            pltpu.SemaphoreType.DMA,
            pltpu.SemaphoreType.DMA,
        ],
        compiler_params=pltpu.CompilerParams(needs_layout_passes=False),
    )


def _cast_kernel(fc1_ref, fc2_ref, o1_ref, o2_ref):
    o1_ref[...] = fc1_ref[...].astype(jnp.bfloat16)
    o2_ref[...] = fc2_ref[...].astype(jnp.bfloat16)


def _cast_weights(fc1_w, fc2_w):
    nh = HIDDEN // D_MODEL
    return pl.pallas_call(
        _cast_kernel,
        grid=(NUM_EXPERTS, nh),
        in_specs=[
            pl.BlockSpec((1, D_MODEL, D_MODEL), lambda e, h: (e, h, 0)),
            pl.BlockSpec((1, D_MODEL, D_MODEL), lambda e, h: (e, 0, h)),
        ],
        out_specs=[
            pl.BlockSpec((1, D_MODEL, D_MODEL), lambda e, h: (e, h, 0)),
            pl.BlockSpec((1, D_MODEL, D_MODEL), lambda e, h: (e, 0, h)),
        ],
        out_shape=[
            jax.ShapeDtypeStruct((NUM_EXPERTS, HIDDEN, D_MODEL), jnp.bfloat16),
            jax.ShapeDtypeStruct((NUM_EXPERTS, D_MODEL, HIDDEN), jnp.bfloat16),
        ],
    )(fc1_w, fc2_w)


def _ffn_kernel(te_ref, gtok_ref, fc1_ref, fc1b_ref, fc2_ref,
                fc2b_ref, out_ref):
    toks = gtok_ref[...].astype(jnp.bfloat16)
    hid = jnp.dot(toks, fc1_ref[0].T, preferred_element_type=jnp.float32)
    hid = hid + fc1b_ref[0]
    hid = 0.5 * hid * (1.0 + lax.erf(hid * 0.7071067811865476))
    o = jnp.dot(hid.astype(jnp.bfloat16), fc2_ref[0].T,
                preferred_element_type=jnp.float32)
    out_ref[...] = o + fc2b_ref[0]


def _ffn(te, gtoks, fc1_bf, fc1_b3, fc2_bf, fc2_b3):
    grid_spec = pltpu.PrefetchScalarGridSpec(
        num_scalar_prefetch=1,
        grid=(NT,),
        in_specs=[
            pl.BlockSpec((TM, D_MODEL), lambda t, te: (t, 0)),
            pl.BlockSpec((1, HIDDEN, D_MODEL), lambda t, te: (te[t], 0, 0)),
            pl.BlockSpec((1, 1, HIDDEN), lambda t, te: (te[t], 0, 0)),
            pl.BlockSpec((1, D_MODEL, HIDDEN), lambda t, te: (te[t], 0, 0)),
            pl.BlockSpec((1, 1, D_MODEL), lambda t, te: (te[t], 0, 0)),
        ],
        out_specs=pl.BlockSpec((TM, D_MODEL), lambda t, te: (t, 0)),
    )
    return pl.pallas_call(
        _ffn_kernel,
        grid_spec=grid_spec,
        out_shape=jax.ShapeDtypeStruct((P_SLOTS, D_MODEL), jnp.float32),
    )(te, gtoks, fc1_bf, fc1_b3, fc2_bf, fc2_b3)


def _combine_kernel(x_ref, op_ref, w_ref, out_ref):
    wv = w_ref[...]
    wcol = lax.broadcasted_iota(jnp.int32, wv.shape, 1)
    w0 = jnp.sum(jnp.where(wcol == 0, wv, 0.0), axis=1, keepdims=True)
    w1 = jnp.sum(jnp.where(wcol == 1, wv, 0.0), axis=1, keepdims=True)
    out_ref[...] = x_ref[...] + w0 * op_ref[0] + w1 * op_ref[1]


def _combine(xf, op_k3, w_tok):
    return pl.pallas_call(
        _combine_kernel,
        grid=(N_TOK // BT,),
        in_specs=[
            pl.BlockSpec((BT, D_MODEL), lambda t: (t, 0)),
            pl.BlockSpec((2, BT, D_MODEL), lambda t: (0, t, 0)),
            pl.BlockSpec((BT, 2), lambda t: (t, 0)),
        ],
        out_specs=pl.BlockSpec((BT, D_MODEL), lambda t: (t, 0)),
        out_shape=jax.ShapeDtypeStruct((N_TOK, D_MODEL), jnp.float32),
    )(xf, op_k3, w_tok)


def kernel(x, rms_weight, gate_w, fc1_w, fc1_b, fc2_w, fc2_b):
    b, s, d = x.shape
    xf = x.reshape(N_TOK, d)
    fc1_bf, fc2_bf = _cast_weights(fc1_w, fc2_w)
    fc1_b3 = fc1_b.reshape(NUM_EXPERTS, 1, HIDDEN)
    fc2_b3 = fc2_b.reshape(NUM_EXPERTS, 1, D_MODEL)

    tok_f, eidx_k, rank_k, w_tok, base, te = _route(xf, rms_weight, gate_w)

    pos_k, grows = _dispatch(eidx_k, rank_k, base.reshape(NUM_EXPERTS),
                             tok_f.reshape(N_TOK, 8, 128))

    out_pairs = _ffn(te.reshape(NT), grows.reshape(P_SLOTS, D_MODEL),
                     fc1_bf, fc1_b3, fc2_bf, fc2_b3)

    op_k = _make_gather(N_PAIR, C_CHUNK, jnp.float32)(
        pos_k.reshape(N_PAIR), out_pairs.reshape(P_SLOTS, 8, 128))

    out = _combine(xf, op_k.reshape(2, N_TOK, D_MODEL), w_tok)
    return out.reshape(b, s, d)


# speedup vs baseline: 1.4696x; 1.1271x over previous
import functools

import jax
import jax.numpy as jnp
from jax import lax
from jax.experimental import pallas as pl
from jax.experimental.pallas import tpu as pltpu
from jax.experimental.pallas import tpu_sc as plsc

D_MODEL = 1024
HIDDEN = 4096
NUM_EXPERTS = 16
EPS = 1e-6

N_TOK = 8192
N_PAIR = 2 * N_TOK
BT = 1024
BTA = 256
TM = 256
P_SLOTS = N_PAIR + NUM_EXPERTS * TM
NT = P_SLOTS // TM

NW = 32
NTW = N_TOK // NW
D_CHUNK = 32
C_CHUNK = 32


def _route_kernel(x_ref, rmsw_ref, gate_ref, tok_ref, eidx_ref, rank_ref,
                  w_ref, base_ref, te_ref, run_cnt):
    t = pl.program_id(0)

    @pl.when(t == 0)
    def _init():
        run_cnt[...] = jnp.zeros_like(run_cnt)

    x = x_ref[...]
    nrm = x * lax.rsqrt(jnp.mean(jnp.square(x), axis=-1, keepdims=True) + EPS)
    nrm = nrm * rmsw_ref[...][None, :]
    tok_ref[...] = nrm

    logits = jnp.dot(nrm, gate_ref[...].T, preferred_element_type=jnp.float32)
    m0 = jnp.max(logits, axis=-1, keepdims=True)
    i0 = jnp.argmax(logits, axis=-1).astype(jnp.int32)
    col = lax.broadcasted_iota(jnp.int32, logits.shape, 1)
    masked = jnp.where(col == i0[:, None], -jnp.inf, logits)
    m1 = jnp.max(masked, axis=-1, keepdims=True)
    i1 = jnp.argmax(masked, axis=-1).astype(jnp.int32)
    w0 = 1.0 / (1.0 + jnp.exp(m1 - m0))
    w1 = 1.0 - w0
    eidx_ref[...] = jnp.concatenate(
        [i0.reshape(1, BTA), i1.reshape(1, BTA)], axis=0)
    w_ref[...] = jnp.concatenate([w0, w1], axis=1)

    oh0 = (col == i0[:, None]).astype(jnp.float32)
    oh1 = (col == i1[:, None]).astype(jnp.float32)
    c = oh0 + oh1
    row = lax.broadcasted_iota(jnp.int32, (BTA, BTA), 0)
    cidx = lax.broadcasted_iota(jnp.int32, (BTA, BTA), 1)
    lt = (cidx < row).astype(jnp.float32)
    excl = jnp.dot(lt, c, preferred_element_type=jnp.float32)
    run = run_cnt[...].astype(jnp.float32)
    tot = excl + run
    rank0 = jnp.sum(tot * oh0, axis=1, keepdims=True).astype(jnp.int32)
    rank1 = jnp.sum(tot * oh1, axis=1, keepdims=True).astype(jnp.int32)
    rank_ref[...] = jnp.concatenate(
        [rank0.reshape(1, BTA), rank1.reshape(1, BTA)], axis=0)
    run_cnt[...] += jnp.sum(c, axis=0, keepdims=True).astype(jnp.int32)

    cnt = run_cnt[...]
    padded = ((cnt + (TM - 1)) // TM) * TM
    pr = lax.broadcasted_iota(jnp.int32, (NUM_EXPERTS, NUM_EXPERTS), 0)
    pc = lax.broadcasted_iota(jnp.int32, (NUM_EXPERTS, NUM_EXPERTS), 1)
    contrib = jnp.where(pr < pc, jnp.broadcast_to(
        padded.reshape(NUM_EXPERTS, 1), (NUM_EXPERTS, NUM_EXPERTS)), 0)
    base = jnp.sum(contrib, axis=0).reshape(1, NUM_EXPERTS)
    base_ref[...] = base
    ends = (base + padded).reshape(NUM_EXPERTS, 1)
    jt = lax.broadcasted_iota(jnp.int32, (NUM_EXPERTS, NT), 1) * TM
    te = jnp.sum((ends <= jt).astype(jnp.int32), axis=0).reshape(1, NT)
    te_ref[...] = jnp.minimum(te, NUM_EXPERTS - 1)


def _route(xf, rms_weight, gate_w):
    return pl.pallas_call(
        _route_kernel,
        grid=(N_TOK // BTA,),
        in_specs=[
            pl.BlockSpec((BTA, D_MODEL), lambda t: (t, 0)),
            pl.BlockSpec((D_MODEL,), lambda t: (0,)),
            pl.BlockSpec((NUM_EXPERTS, D_MODEL), lambda t: (0, 0)),
        ],
        out_specs=[
            pl.BlockSpec((BTA, D_MODEL), lambda t: (t, 0)),
            pl.BlockSpec((2, BTA), lambda t: (0, t)),
            pl.BlockSpec((2, BTA), lambda t: (0, t)),
            pl.BlockSpec((BTA, 2), lambda t: (t, 0)),
            pl.BlockSpec((1, NUM_EXPERTS), lambda t: (0, 0)),
            pl.BlockSpec((1, NT), lambda t: (0, 0)),
        ],
        out_shape=[
            jax.ShapeDtypeStruct((N_TOK, D_MODEL), jnp.float32),
            jax.ShapeDtypeStruct((2, N_TOK), jnp.int32),
            jax.ShapeDtypeStruct((2, N_TOK), jnp.int32),
            jax.ShapeDtypeStruct((N_TOK, 2), jnp.float32),
            jax.ShapeDtypeStruct((1, NUM_EXPERTS), jnp.int32),
            jax.ShapeDtypeStruct((1, NT), jnp.int32),
        ],
        scratch_shapes=[pltpu.VMEM((1, NUM_EXPERTS), jnp.int32)],
    )(xf, rms_weight, gate_w)


_N_DCHUNKS = NTW // D_CHUNK


def _dispatch_body(eidx_hbm, rank_hbm, base_hbm, tok_hbm,
                   pos_hbm, grows_hbm,
                   ve0, ve1, vr0, vr1, vbase, vp0, vp1, vp0c, vp1c,
                   row0, row1, ls0, ls1, ss0, ss1):
    cid = lax.axis_index("c")
    sid = lax.axis_index("s")
    wid = sid * 2 + cid
    toff = wid * NTW

    pltpu.sync_copy(eidx_hbm.at[0, pl.ds(toff, NTW)], ve0)
    pltpu.sync_copy(eidx_hbm.at[1, pl.ds(toff, NTW)], ve1)
    pltpu.sync_copy(rank_hbm.at[0, pl.ds(toff, NTW)], vr0)
    pltpu.sync_copy(rank_hbm.at[1, pl.ds(toff, NTW)], vr1)
    pltpu.sync_copy(base_hbm, vbase)

    for g in range(NTW // 16):
        c, o = g // (D_CHUNK // 16), (g % (D_CHUNK // 16)) * 16
        p0 = vr0[pl.ds(g * 16, 16)] + plsc.load_gather(
            vbase, [ve0[pl.ds(g * 16, 16)]])
        vp0[pl.ds(g * 16, 16)] = p0
        vp0c[c, pl.ds(o, 16)] = p0
        p1 = vr1[pl.ds(g * 16, 16)] + plsc.load_gather(
            vbase, [ve1[pl.ds(g * 16, 16)]])
        vp1[pl.ds(g * 16, 16)] = p1
        vp1c[c, pl.ds(o, 16)] = p1

    pltpu.sync_copy(vp0, pos_hbm.at[0, pl.ds(toff, NTW)])
    pltpu.sync_copy(vp1, pos_hbm.at[1, pl.ds(toff, NTW)])

    bufs = (row0, row1)
    lsems = (ls0, ls1)
    ssems = (ss0, ss1)

    def load(c):
        return pltpu.async_copy(
            tok_hbm.at[pl.ds(toff + c * D_CHUNK, D_CHUNK)],
            bufs[c % 2], lsems[c % 2])

    ld = [None] * _N_DCHUNKS
    sc = [None] * _N_DCHUNKS
    ld[0] = load(0)
    ld[1] = load(1)
    for c in range(_N_DCHUNKS):
        ld[c].wait()
        s0 = pltpu.async_copy(bufs[c % 2], grows_hbm.at[vp0c.at[c]],
                              ssems[c % 2])
        s1 = pltpu.async_copy(bufs[c % 2], grows_hbm.at[vp1c.at[c]],
                              ssems[c % 2])
        sc[c] = (s0, s1)
        nxt = c + 2
        if nxt < _N_DCHUNKS:
            sc[c][0].wait()
            sc[c][1].wait()
            ld[nxt] = load(nxt)
    for c in (_N_DCHUNKS - 2, _N_DCHUNKS - 1):
        sc[c][0].wait()
        sc[c][1].wait()


def _dispatch(eidx_k, rank_k, base_f, tok3):
    mesh = plsc.VectorSubcoreMesh(core_axis_name="c", subcore_axis_name="s")
    fn = pl.kernel(
        _dispatch_body,
        out_type=[
            jax.ShapeDtypeStruct((2, N_TOK), jnp.int32),
            jax.ShapeDtypeStruct((P_SLOTS, 8, 128), jnp.float32),
        ],
        mesh=mesh,
        scratch_types=[
            pltpu.VMEM((NTW,), jnp.int32),
            pltpu.VMEM((NTW,), jnp.int32),
            pltpu.VMEM((NTW,), jnp.int32),
            pltpu.VMEM((NTW,), jnp.int32),
            pltpu.VMEM((NUM_EXPERTS,), jnp.int32),
            pltpu.VMEM((NTW,), jnp.int32),
            pltpu.VMEM((NTW,), jnp.int32),
            pltpu.VMEM((_N_DCHUNKS, D_CHUNK), jnp.int32),
            pltpu.VMEM((_N_DCHUNKS, D_CHUNK), jnp.int32),
            pltpu.VMEM((D_CHUNK, 8, 128), jnp.float32),
            pltpu.VMEM((D_CHUNK, 8, 128), jnp.float32),
            pltpu.SemaphoreType.DMA,
            pltpu.SemaphoreType.DMA,
            pltpu.SemaphoreType.DMA,
            pltpu.SemaphoreType.DMA,
        ],
        compiler_params=pltpu.CompilerParams(needs_layout_passes=False),
    )
    return fn(eidx_k, rank_k, base_f, tok3)


def _make_gather(n_rows, n_chunk, dtype, sl=8):
    rows_per_w = n_rows // NW
    n_loops = rows_per_w // n_chunk
    assert rows_per_w % n_chunk == 0 and n_loops >= 2

    def body(idx_hbm, table_hbm, out_hbm, idx_v, rows0, rows1,
             gs0, gs1, os0, os1):
        cid = lax.axis_index("c")
        sid = lax.axis_index("s")
        wid = sid * 2 + cid
        base = wid * rows_per_w
        pltpu.sync_copy(idx_hbm.at[pl.ds(base, rows_per_w)], idx_v)

        bufs = (rows0, rows1)
        gsems = (gs0, gs1)
        osems = (os0, os1)
        gd = [None] * n_loops
        od = [None] * n_loops

        def issue_gather(ci):
            return pltpu.async_copy(
                table_hbm.at[idx_v.at[pl.ds(ci * n_chunk, n_chunk)]],
                bufs[ci % 2], gsems[ci % 2])

        gd[0] = issue_gather(0)
        gd[1] = issue_gather(1)
        for ci in range(n_loops):
            gd[ci].wait()
            od[ci] = pltpu.async_copy(
                bufs[ci % 2],
                out_hbm.at[pl.ds(base + ci * n_chunk, n_chunk)],
                osems[ci % 2])
            nxt = ci + 2
            if nxt < n_loops:
                od[ci].wait()
                gd[nxt] = issue_gather(nxt)
        od[n_loops - 2].wait()
        od[n_loops - 1].wait()

    mesh = plsc.VectorSubcoreMesh(core_axis_name="c", subcore_axis_name="s")
    return pl.kernel(
        body,
        out_type=jax.ShapeDtypeStruct((n_rows, sl, 128), dtype),
        mesh=mesh,
        scratch_types=[
            pltpu.VMEM((rows_per_w,), jnp.int32),
            pltpu.VMEM((n_chunk, sl, 128), dtype),
            pltpu.VMEM((n_chunk, sl, 128), dtype),
            pltpu.SemaphoreType.DMA,
            pltpu.SemaphoreType.DMA,
            pltpu.SemaphoreType.DMA,
            pltpu.SemaphoreType.DMA,
        ],
        compiler_params=pltpu.CompilerParams(needs_layout_passes=False),
    )


def _cast_kernel(fc1_ref, fc2_ref, o1_ref, o2_ref):
    o1_ref[...] = fc1_ref[...].astype(jnp.bfloat16)
    o2_ref[...] = fc2_ref[...].astype(jnp.bfloat16)


def _cast_weights(fc1_w, fc2_w):
    nh = HIDDEN // D_MODEL
    return pl.pallas_call(
        _cast_kernel,
        grid=(NUM_EXPERTS, nh),
        in_specs=[
            pl.BlockSpec((1, D_MODEL, D_MODEL), lambda e, h: (e, h, 0)),
            pl.BlockSpec((1, D_MODEL, D_MODEL), lambda e, h: (e, 0, h)),
        ],
        out_specs=[
            pl.BlockSpec((1, D_MODEL, D_MODEL), lambda e, h: (e, h, 0)),
            pl.BlockSpec((1, D_MODEL, D_MODEL), lambda e, h: (e, 0, h)),
        ],
        out_shape=[
            jax.ShapeDtypeStruct((NUM_EXPERTS, HIDDEN, D_MODEL), jnp.bfloat16),
            jax.ShapeDtypeStruct((NUM_EXPERTS, D_MODEL, HIDDEN), jnp.bfloat16),
        ],
    )(fc1_w, fc2_w)


def _ffn_kernel(te_ref, gtok_ref, fc1_ref, fc1b_ref, fc2_ref,
                fc2b_ref, out_ref):
    toks = gtok_ref[...].reshape(TM, D_MODEL).astype(jnp.bfloat16)
    hid = jnp.dot(toks, fc1_ref[0].T, preferred_element_type=jnp.float32)
    hid = hid + fc1b_ref[0]
    hid = 0.5 * hid * (1.0 + lax.erf(hid * 0.7071067811865476))
    o = jnp.dot(hid.astype(jnp.bfloat16), fc2_ref[0].T,
                preferred_element_type=jnp.float32)
    out_ref[...] = (o + fc2b_ref[0]).reshape(TM, 8, 128)


def _ffn(te, gtoks, fc1_bf, fc1_b3, fc2_bf, fc2_b3):
    grid_spec = pltpu.PrefetchScalarGridSpec(
        num_scalar_prefetch=1,
        grid=(NT,),
        in_specs=[
            pl.BlockSpec((TM, 8, 128), lambda t, te: (t, 0, 0)),
            pl.BlockSpec((1, HIDDEN, D_MODEL), lambda t, te: (te[t], 0, 0)),
            pl.BlockSpec((1, 1, HIDDEN), lambda t, te: (te[t], 0, 0)),
            pl.BlockSpec((1, D_MODEL, HIDDEN), lambda t, te: (te[t], 0, 0)),
            pl.BlockSpec((1, 1, D_MODEL), lambda t, te: (te[t], 0, 0)),
        ],
        out_specs=pl.BlockSpec((TM, 8, 128), lambda t, te: (t, 0, 0)),
    )
    return pl.pallas_call(
        _ffn_kernel,
        grid_spec=grid_spec,
        out_shape=jax.ShapeDtypeStruct((P_SLOTS, 8, 128), jnp.float32),
    )(te, gtoks, fc1_bf, fc1_b3, fc2_bf, fc2_b3)


def _combine_kernel(x_ref, op_ref, w_ref, out_ref):
    wv = w_ref[...]
    wcol = lax.broadcasted_iota(jnp.int32, wv.shape, 1)
    w0 = jnp.sum(jnp.where(wcol == 0, wv, 0.0), axis=1, keepdims=True)
    w1 = jnp.sum(jnp.where(wcol == 1, wv, 0.0), axis=1, keepdims=True)
    out_ref[...] = x_ref[...] + w0 * op_ref[0] + w1 * op_ref[1]


def _combine(xf, op_k3, w_tok):
    return pl.pallas_call(
        _combine_kernel,
        grid=(N_TOK // BT,),
        in_specs=[
            pl.BlockSpec((BT, D_MODEL), lambda t: (t, 0)),
            pl.BlockSpec((2, BT, D_MODEL), lambda t: (0, t, 0)),
            pl.BlockSpec((BT, 2), lambda t: (t, 0)),
        ],
        out_specs=pl.BlockSpec((BT, D_MODEL), lambda t: (t, 0)),
        out_shape=jax.ShapeDtypeStruct((N_TOK, D_MODEL), jnp.float32),
    )(xf, op_k3, w_tok)


def kernel(x, rms_weight, gate_w, fc1_w, fc1_b, fc2_w, fc2_b):
    b, s, d = x.shape
    xf = x.reshape(N_TOK, d)
    fc1_bf, fc2_bf = _cast_weights(fc1_w, fc2_w)
    fc1_b3 = fc1_b.reshape(NUM_EXPERTS, 1, HIDDEN)
    fc2_b3 = fc2_b.reshape(NUM_EXPERTS, 1, D_MODEL)

    tok_f, eidx_k, rank_k, w_tok, base, te = _route(xf, rms_weight, gate_w)

    pos_k, grows = _dispatch(eidx_k, rank_k, base.reshape(NUM_EXPERTS),
                             tok_f.reshape(N_TOK, 8, 128))

    out_pairs = _ffn(te.reshape(NT), grows, fc1_bf, fc1_b3, fc2_bf, fc2_b3)

    op_k = _make_gather(N_PAIR, C_CHUNK, jnp.float32)(
        pos_k.reshape(N_PAIR), out_pairs)

    out = _combine(xf, op_k.reshape(2, N_TOK, D_MODEL), w_tok)
    return out.reshape(b, s, d)
